# baseline (speedup 1.0000x reference)
import functools

import jax
import jax.numpy as jnp
from jax import lax
from jax.experimental import pallas as pl
from jax.experimental.pallas import tpu as pltpu
from jax.experimental.pallas import tpu_sc as plsc

_L = 16
_NS = 16
_NC = 2
_NW = _NC * _NS
_ROWS = 16384
_CHUNK = _ROWS // _NW
_ITERS = _CHUNK // _L


def _fprate_sc(out_hbm, tgt_hbm, part_hbm, out_v, tgt_v, part_v):
    cid = lax.axis_index("c")
    sid = lax.axis_index("s")
    wid = cid * _NS + sid

    base = wid * _CHUNK
    pltpu.sync_copy(out_hbm.at[pl.ds(2 * base, 2 * _CHUNK)], out_v)
    pltpu.sync_copy(tgt_hbm.at[pl.ds(base, _CHUNK)], tgt_v)

    lanes = lax.broadcasted_iota(jnp.int32, (_L,), 0)
    one = jnp.ones((_L,), jnp.int32)
    zero = jnp.zeros((_L,), jnp.int32)

    acc_fp = zero
    acc_n0 = zero
    for j in range(_ITERS):
        even = 2 * _L * j + 2 * lanes
        c0 = plsc.load_gather(out_v, [even])
        c1 = plsc.load_gather(out_v, [even + 1])
        t0 = tgt_v[pl.ds(j * _L, _L)] == 0
        acc_fp = acc_fp + jnp.where((c1 > c0) & t0, one, zero)
        acc_n0 = acc_n0 + jnp.where(t0, one, zero)

    part_v[0, :] = acc_fp.astype(jnp.float32)
    part_v[1, :] = acc_n0.astype(jnp.float32)
    pltpu.sync_copy(part_v, part_hbm.at[wid])


def _combine_body(part_ref, res_ref):
    fp = jnp.sum(part_ref[:, 0, :])
    n0 = jnp.sum(part_ref[:, 1, :])
    res_ref[...] = (fp / (n0 + 1e-10)).reshape(1, 1)


def kernel(output, target):
    target = target.astype(jnp.int32)
    count = functools.partial(
        pl.kernel,
        mesh=plsc.VectorSubcoreMesh(core_axis_name="c", subcore_axis_name="s"),
        compiler_params=pltpu.CompilerParams(needs_layout_passes=False),
        out_type=jax.ShapeDtypeStruct((_NW, 2, _L), jnp.float32),
        scratch_types=[
            pltpu.VMEM((2 * _CHUNK,), jnp.float32),
            pltpu.VMEM((_CHUNK,), jnp.int32),
            pltpu.VMEM((2, _L), jnp.float32),
        ],
    )(_fprate_sc)
    parts = count(output.reshape(-1), target)
    res = pl.pallas_call(
        _combine_body,
        out_shape=jax.ShapeDtypeStruct((1, 1), jnp.float32),
    )(parts)
    return res[0, 0]


# speedup vs baseline: 1.0611x; 1.0611x over previous
import functools

import jax
import jax.numpy as jnp
from jax import lax
from jax.experimental import pallas as pl
from jax.experimental.pallas import tpu as pltpu
from jax.experimental.pallas import tpu_sc as plsc

_L = 16
_NW = 16
_ROWS = 16384
_CHUNK = _ROWS // _NW
_ITERS = _CHUNK // _L


def _fprate_sc(out_hbm, tgt_hbm, part_hbm, out_v, tgt_v, part_v, sem1, sem2):
    sid = lax.axis_index("s")

    base = sid * _CHUNK
    cp_out = pltpu.make_async_copy(
        out_hbm.at[pl.ds(2 * base, 2 * _CHUNK)], out_v, sem1)
    cp_tgt = pltpu.make_async_copy(
        tgt_hbm.at[pl.ds(base, _CHUNK)], tgt_v, sem2)
    cp_out.start()
    cp_tgt.start()
    cp_out.wait()
    cp_tgt.wait()

    lanes = lax.broadcasted_iota(jnp.int32, (_L,), 0)
    one = jnp.ones((_L,), jnp.int32)
    zero = jnp.zeros((_L,), jnp.int32)

    acc_fp = zero
    acc_n0 = zero
    for j in range(_ITERS):
        even = 2 * _L * j + 2 * lanes
        c0 = plsc.load_gather(out_v, [even])
        c1 = plsc.load_gather(out_v, [even + 1])
        t0 = tgt_v[pl.ds(j * _L, _L)] == 0
        acc_fp = acc_fp + jnp.where((c1 > c0) & t0, one, zero)
        acc_n0 = acc_n0 + jnp.where(t0, one, zero)

    part_v[0, :] = acc_fp.astype(jnp.float32)
    part_v[1, :] = acc_n0.astype(jnp.float32)
    pltpu.sync_copy(part_v, part_hbm.at[sid])


def _combine_body(part_ref, res_ref):
    fp = jnp.sum(part_ref[:, 0, :])
    n0 = jnp.sum(part_ref[:, 1, :])
    res_ref[...] = (fp / (n0 + 1e-10)).reshape(1, 1)


def kernel(output, target):
    target = target.astype(jnp.int32)
    count = functools.partial(
        pl.kernel,
        mesh=plsc.VectorSubcoreMesh(core_axis_name="c", subcore_axis_name="s",
                                    num_cores=1),
        compiler_params=pltpu.CompilerParams(needs_layout_passes=False),
        out_type=jax.ShapeDtypeStruct((_NW, 2, _L), jnp.float32),
        scratch_types=[
            pltpu.VMEM((2 * _CHUNK,), jnp.float32),
            pltpu.VMEM((_CHUNK,), jnp.int32),
            pltpu.VMEM((2, _L), jnp.float32),
            pltpu.SemaphoreType.DMA,
            pltpu.SemaphoreType.DMA,
        ],
    )(_fprate_sc)
    parts = count(output.reshape(-1), target)
    res = pl.pallas_call(
        _combine_body,
        out_shape=jax.ShapeDtypeStruct((1, 1), jnp.float32),
    )(parts)
    return res[0, 0]


# speedup vs baseline: 1.0620x; 1.0008x over previous
import functools

import jax
import jax.numpy as jnp
from jax import lax
from jax.experimental import pallas as pl
from jax.experimental.pallas import tpu as pltpu
from jax.experimental.pallas import tpu_sc as plsc

_L = 16
_NW = 16
_ROWS = 16384
_CHUNK = _ROWS // _NW
_ITERS = _CHUNK // _L


def _fprate_sc(out_hbm, tgt_hbm, part_hbm, out_v, tgt_v, part_v, sem1, sem2):
    sid = lax.axis_index("s")

    base = sid * _CHUNK
    cp_out = pltpu.make_async_copy(
        out_hbm.at[pl.ds(2 * base, 2 * _CHUNK)], out_v, sem1)
    cp_tgt = pltpu.make_async_copy(
        tgt_hbm.at[pl.ds(base, _CHUNK)], tgt_v, sem2)
    cp_out.start()
    cp_tgt.start()
    cp_out.wait()
    cp_tgt.wait()

    lanes = lax.broadcasted_iota(jnp.int32, (_L,), 0)
    one = jnp.ones((_L,), jnp.int32)
    zero = jnp.zeros((_L,), jnp.int32)

    acc_fp = zero
    acc_t1 = zero
    for j in range(_ITERS):
        even = 2 * _L * j + 2 * lanes
        c0 = plsc.load_gather(out_v, [even])
        c1 = plsc.load_gather(out_v, [even + 1])
        t = tgt_v[pl.ds(j * _L, _L)]
        acc_fp = acc_fp + jnp.where(c1 > c0, one - t, zero)
        acc_t1 = acc_t1 + t

    part_v[0, :] = acc_fp.astype(jnp.float32)
    part_v[1, :] = (_ITERS - acc_t1).astype(jnp.float32)
    pltpu.sync_copy(part_v, part_hbm.at[sid])


def _combine_body(part_ref, res_ref):
    fp = jnp.sum(part_ref[:, 0, :])
    n0 = jnp.sum(part_ref[:, 1, :])
    res_ref[...] = (fp / (n0 + 1e-10)).reshape(1, 1)


def kernel(output, target):
    target = target.astype(jnp.int32)
    count = functools.partial(
        pl.kernel,
        mesh=plsc.VectorSubcoreMesh(core_axis_name="c", subcore_axis_name="s",
                                    num_cores=1),
        compiler_params=pltpu.CompilerParams(needs_layout_passes=False),
        out_type=jax.ShapeDtypeStruct((_NW, 2, _L), jnp.float32),
        scratch_types=[
            pltpu.VMEM((2 * _CHUNK,), jnp.float32),
            pltpu.VMEM((_CHUNK,), jnp.int32),
            pltpu.VMEM((2, _L), jnp.float32),
            pltpu.SemaphoreType.DMA,
            pltpu.SemaphoreType.DMA,
        ],
    )(_fprate_sc)
    parts = count(output.reshape(-1), target)
    res = pl.pallas_call(
        _combine_body,
        out_shape=jax.ShapeDtypeStruct((1, 1), jnp.float32),
    )(parts)
    return res[0, 0]
